# Initial kernel scaffold; baseline (speedup 1.0000x reference)
import jax
import jax.numpy as jnp
from jax.experimental import pallas as pl


def kernel(x, edge_index, original_edge_index, batch, num_subgraphs, num_nodes_per_subgraph, subgraph_batch, subgraph_node_idx, subgraph_idx_batch, gWs, gWn, gb, bn_g, bn_b, sWs, sWn, sb, bn2_g, bn2_b, aWq, aWk, abq, abk, W1, b1, W2, b2):
    raise NotImplementedError("write your pallas kernel here")


# speedup vs baseline: 8.2557x; 8.2557x over previous
import functools
import math

import jax
import jax.numpy as jnp
from jax import lax
from jax.experimental import pallas as pl
from jax.experimental.pallas import tpu as pltpu
from jax.experimental.pallas import tpu_sc as plsc

G = 16
S = 40
NPS = 40
N = G * S * NPS
D = 128
DH = D // 2
E = 409600
ON = G * NPS
EO = 10240
GS = G * S

NTILES = 16
EPT = E // NTILES
CH = 128
NCH = EPT // CH
ZROWS = N // NTILES


DQ = D // 4


def _sc_segsum(x4, src_r, dst_r, zrows):
    mesh = plsc.VectorSubcoreMesh(core_axis_name="c", subcore_axis_name="s")

    @functools.partial(
        pl.kernel,
        out_type=jax.ShapeDtypeStruct((4, N, DQ), jnp.float32),
        mesh=mesh,
        scratch_types=[
            pltpu.VMEM((NCH, CH), jnp.int32),
            pltpu.VMEM((NCH, CH), jnp.int32),
            pltpu.VMEM((CH, DQ), jnp.float32),
            pltpu.VMEM((CH, DQ), jnp.float32),
            pltpu.VMEM_SHARED((N, DQ), jnp.float32),
            pltpu.SemaphoreType.DMA,
            pltpu.SemaphoreType.DMA,
            pltpu.SemaphoreType.DMA,
        ],
        compiler_params=pltpu.CompilerParams(use_tc_tiling_on_sc=False),
    )
    def segsum(x4_hbm, src_hbm, dst_hbm, z_hbm, out_hbm,
               srcv, dstv, rows_a, rows_b, acc, gsem_a, gsem_b, ssem):
        c = lax.axis_index("c")
        s = lax.axis_index("s")

        pltpu.sync_copy(src_hbm.at[s], srcv)
        pltpu.sync_copy(dst_hbm.at[s], dstv)

        @pl.loop(0, NCH)
        def _(j):
            @pl.loop(0, CH, step=16)
            def _(k):
                v = srcv[j, pl.ds(k, 16)]
                srcv[j, pl.ds(k, 16)] = v * 4 + 2 * c

        for p in range(2):
            pltpu.sync_copy(z_hbm, acc.at[pl.ds(s * ZROWS, ZROWS)])
            plsc.subcore_barrier()

            pltpu.async_copy(x4_hbm.at[srcv.at[0]], rows_a, gsem_a)
            pltpu.async_copy(x4_hbm.at[srcv.at[1]], rows_b, gsem_b)

            @pl.loop(0, NCH, step=2)
            def _(j):
                pltpu.make_async_copy(
                    x4_hbm.at[srcv.at[j]], rows_a, gsem_a).wait()
                cp_a = pltpu.async_copy(rows_a, acc.at[dstv.at[j]], ssem,
                                        add=True)
                pltpu.make_async_copy(
                    x4_hbm.at[srcv.at[j]], rows_b, gsem_b).wait()
                cp_a.wait()

                @pl.when(j + 2 < NCH)
                def _():
                    pltpu.async_copy(x4_hbm.at[srcv.at[j + 2]], rows_a, gsem_a)

                cp_b = pltpu.async_copy(rows_b, acc.at[dstv.at[j + 1]], ssem,
                                        add=True)
                cp_b.wait()

                @pl.when(j + 3 < NCH)
                def _():
                    pltpu.async_copy(x4_hbm.at[srcv.at[j + 3]], rows_b, gsem_b)

            plsc.subcore_barrier()
            pltpu.sync_copy(acc.at[pl.ds(s * ZROWS, ZROWS)],
                            out_hbm.at[2 * c + p, pl.ds(s * ZROWS, ZROWS)])

            if p == 0:
                plsc.subcore_barrier()

                @pl.loop(0, NCH)
                def _(j):
                    @pl.loop(0, CH, step=16)
                    def _(k):
                        srcv[j, pl.ds(k, 16)] = srcv[j, pl.ds(k, 16)] + 1

    return segsum(x4, src_r, dst_r, zrows)


BR = N // G


def _adj_kernel(s_ref, d_ref, adj_ref):
    i = pl.program_id(0)

    @pl.when(i == 0)
    def _():
        adj_ref[...] = jnp.zeros_like(adj_ref)

    cols = lax.broadcasted_iota(jnp.int32, (EO // 8, ON), 1)
    oh_s = (s_ref[0, 0, :][:, None] == cols).astype(jnp.bfloat16)
    oh_d = (d_ref[0, 0, :][:, None] == cols).astype(jnp.bfloat16)
    adj_ref[...] += lax.dot_general(
        oh_d, oh_s, (((0,), (0,)), ((), ())),
        preferred_element_type=jnp.float32)


def _build_adj(oe):
    src = oe[0].reshape(8, 1, EO // 8)
    dst = oe[1].reshape(8, 1, EO // 8)
    return pl.pallas_call(
        _adj_kernel,
        grid=(8,),
        in_specs=[
            pl.BlockSpec((1, 1, EO // 8), lambda i: (i, 0, 0)),
            pl.BlockSpec((1, 1, EO // 8), lambda i: (i, 0, 0)),
        ],
        out_specs=pl.BlockSpec((ON, ON), lambda i: (0, 0)),
        out_shape=jax.ShapeDtypeStruct((ON, ON), jnp.float32),
    )(src, dst)


def _xws_kernel(x_ref, w_ref, xw_ref, gsum_ref):
    xb = x_ref[...]
    xw_ref[...] = jnp.dot(xb, w_ref[...], preferred_element_type=jnp.float32)
    gsum_ref[...] = xb.reshape(S, NPS, D).sum(axis=1)


def _xws(x, w):
    return pl.pallas_call(
        _xws_kernel,
        grid=(G,),
        in_specs=[
            pl.BlockSpec((BR, D), lambda g: (g, 0)),
            pl.BlockSpec((D, D), lambda g: (0, 0)),
        ],
        out_specs=[
            pl.BlockSpec((BR, D), lambda g: (g, 0)),
            pl.BlockSpec((S, D), lambda g: (g, 0)),
        ],
        out_shape=[
            jax.ShapeDtypeStruct((N, D), jnp.float32),
            jax.ShapeDtypeStruct((GS, D), jnp.float32),
        ],
    )(x, w)


def _t1_kernel(xw_ref, a0_ref, a1_ref, a2_ref, a3_ref,
               w0_ref, w1_ref, w2_ref, w3_ref, b_ref,
               t1_ref, ssum_ref, ssq_ref):
    g = pl.program_id(0)
    t1 = xw_ref[...] + b_ref[...]
    for a_ref, w_ref in ((a0_ref, w0_ref), (a1_ref, w1_ref),
                         (a2_ref, w2_ref), (a3_ref, w3_ref)):
        t1 += jnp.dot(a_ref[0], w_ref[...], preferred_element_type=jnp.float32)
    t1_ref[...] = t1

    @pl.when(g == 0)
    def _():
        ssum_ref[...] = jnp.zeros_like(ssum_ref)
        ssq_ref[...] = jnp.zeros_like(ssq_ref)

    ssum_ref[...] += jnp.sum(t1, axis=0, keepdims=True)
    ssq_ref[...] += jnp.sum(t1 * t1, axis=0, keepdims=True)


def _t1(xw, agg, wn, b):
    return pl.pallas_call(
        _t1_kernel,
        grid=(G,),
        in_specs=(
            [pl.BlockSpec((BR, D), lambda g: (g, 0))]
            + [pl.BlockSpec((1, BR, DQ),
                            functools.partial(lambda q, g: (q, g, 0), q))
               for q in range(4)]
            + [pl.BlockSpec((DQ, D), lambda g: (0, 0)) for _ in range(4)]
            + [pl.BlockSpec((1, D), lambda g: (0, 0))]
        ),
        out_specs=[
            pl.BlockSpec((BR, D), lambda g: (g, 0)),
            pl.BlockSpec((1, D), lambda g: (0, 0)),
            pl.BlockSpec((1, D), lambda g: (0, 0)),
        ],
        out_shape=[
            jax.ShapeDtypeStruct((N, D), jnp.float32),
            jax.ShapeDtypeStruct((1, D), jnp.float32),
            jax.ShapeDtypeStruct((1, D), jnp.float32),
        ],
    )(xw, agg, agg, agg, agg,
      wn[0:DQ], wn[DQ:2 * DQ], wn[2 * DQ:3 * DQ], wn[3 * DQ:4 * DQ],
      b.reshape(1, D))


def _atten_kernel(gsum_ref, x_ref, adj_ref, wq_ref, wk_ref, bq_ref, bk_ref,
                  sws_ref, swn_ref, sb_ref, g2_ref, b2_ref,
                  h2_ref, heat_ref):
    gsf = gsum_ref[...] * (1.0 / NPS)
    q = jnp.dot(gsf, wq_ref[...], preferred_element_type=jnp.float32) + bq_ref[...]
    k = jnp.dot(gsf, wk_ref[...], preferred_element_type=jnp.float32) + bk_ref[...]
    scale = 1.0 / math.sqrt(float(D))
    xa_parts = []
    for g in range(G):
        qg = lax.slice(q, (g * S, 0), ((g + 1) * S, D))
        kg = lax.slice(k, (g * S, 0), ((g + 1) * S, D))
        sc = lax.dot_general(qg, kg, (((1,), (1,)), ((), ())),
                             preferred_element_type=jnp.float32) * scale
        m = jnp.max(sc, axis=-1, keepdims=True)
        ex = jnp.exp(sc - m)
        a = ex / jnp.sum(ex, axis=-1, keepdims=True)
        if g == G - 1:
            heat_ref[...] = a
        xg = x_ref[pl.ds(g * BR, BR), :].reshape(S, NPS, D)
        xa_parts.append(jnp.sum(a[:, :, None] * xg, axis=0))
    xa = jnp.concatenate(xa_parts, axis=0)
    aggo = jnp.dot(adj_ref[...].astype(jnp.float32), xa,
                   preferred_element_type=jnp.float32)
    pre = (jnp.dot(xa, sws_ref[...], preferred_element_type=jnp.float32)
           + jnp.dot(aggo, swn_ref[...], preferred_element_type=jnp.float32)
           + sb_ref[...])
    mu = jnp.mean(pre, axis=0, keepdims=True)
    var = jnp.mean((pre - mu) * (pre - mu), axis=0, keepdims=True)
    h2_ref[...] = ((pre - mu) * lax.rsqrt(var + 1e-5) * g2_ref[...]
                   + b2_ref[...])


def _atten(gsum, x, adj, wq, wk, bq, bk, sws, swn, sb, g2, b2):
    full = lambda shape: pl.BlockSpec(shape, lambda: tuple(0 for _ in shape))
    return pl.pallas_call(
        _atten_kernel,
        in_specs=[
            full((GS, D)), full((N, D)), full((ON, ON)),
            full((D, D)), full((D, D)), full((1, D)), full((1, D)),
            full((D, D)), full((D, D)), full((1, D)),
            full((1, D)), full((1, D)),
        ],
        out_specs=[full((ON, D)), full((S, S))],
        out_shape=[
            jax.ShapeDtypeStruct((ON, D), jnp.float32),
            jax.ShapeDtypeStruct((S, S), jnp.float32),
        ],
    )(gsum, x, adj, wq, wk, bq.reshape(1, D), bk.reshape(1, D),
      sws, swn, sb.reshape(1, D), g2.reshape(1, D), b2.reshape(1, D))


def _combine_kernel(t1_ref, ssum_ref, ssq_ref, g_ref, b_ref, h2_ref,
                    x_ref, hsub_ref):
    mu = ssum_ref[...] * (1.0 / N)
    var = ssq_ref[...] * (1.0 / N) - mu * mu
    sc = lax.rsqrt(var + 1e-5) * g_ref[...]
    sh = b_ref[...] - mu * sc
    t1 = t1_ref[...].reshape(S, NPS, D)
    xn = jnp.maximum(t1 * sc[0][None, None, :] + sh[0][None, None, :]
                     + h2_ref[...][None, :, :], 0.0)
    x_ref[...] = xn.reshape(BR, D)
    hsub_ref[...] = xn.sum(axis=1) * (1.0 / NPS)


def _combine(t1, ssum, ssq, g, b, h2):
    return pl.pallas_call(
        _combine_kernel,
        grid=(G,),
        in_specs=[
            pl.BlockSpec((BR, D), lambda g: (g, 0)),
            pl.BlockSpec((1, D), lambda g: (0, 0)),
            pl.BlockSpec((1, D), lambda g: (0, 0)),
            pl.BlockSpec((1, D), lambda g: (0, 0)),
            pl.BlockSpec((1, D), lambda g: (0, 0)),
            pl.BlockSpec((NPS, D), lambda g: (g, 0)),
        ],
        out_specs=[
            pl.BlockSpec((BR, D), lambda g: (g, 0)),
            pl.BlockSpec((S, D), lambda g: (g, 0)),
        ],
        out_shape=[
            jax.ShapeDtypeStruct((N, D), jnp.float32),
            jax.ShapeDtypeStruct((GS, D), jnp.float32),
        ],
    )(t1, ssum, ssq, g.reshape(1, D), b.reshape(1, D), h2)


def _readout_kernel(hsub_ref, w1_ref, b1_ref, w2_ref, b2_ref, out_ref):
    hg = hsub_ref[...].reshape(G, S, D).mean(axis=1)
    h = jnp.maximum(
        jnp.dot(hg, w1_ref[...], preferred_element_type=jnp.float32)
        + b1_ref[...], 0.0)
    out_ref[...] = (jnp.dot(h, w2_ref[...], preferred_element_type=jnp.float32)
                    + b2_ref[...])


def _readout(hsub, w1, b1, w2, b2, nt):
    full = lambda shape: pl.BlockSpec(shape, lambda: tuple(0 for _ in shape))
    return pl.pallas_call(
        _readout_kernel,
        in_specs=[full((GS, D)), full((D, 2 * D)), full((1, 2 * D)),
                  full((2 * D, nt)), full((1, nt))],
        out_specs=full((G, nt)),
        out_shape=jax.ShapeDtypeStruct((G, nt), jnp.float32),
    )(hsub, w1, b1.reshape(1, 2 * D), w2, b2.reshape(1, nt))


def kernel(x, edge_index, original_edge_index, batch, num_subgraphs,
           num_nodes_per_subgraph, subgraph_batch, subgraph_node_idx,
           subgraph_idx_batch, gWs, gWn, gb, bn_g, bn_b, sWs, sWn, sb,
           bn2_g, bn2_b, aWq, aWk, abq, abk, W1, b1, W2, b2):
    L = gWs.shape[0]
    nt = W2.shape[1]
    src_r = edge_index[0].astype(jnp.int32).reshape(NTILES, NCH, CH)
    dst_r = edge_index[1].astype(jnp.int32).reshape(NTILES, NCH, CH)
    zrows = jnp.zeros((ZROWS, DQ), jnp.float32)
    adj = _build_adj(original_edge_index.astype(jnp.int32))

    heat = None
    hsub = None
    for i in range(L):
        agg = _sc_segsum(x.reshape(4 * N, DQ), src_r, dst_r, zrows)
        xw, gsum = _xws(x, gWs[i])
        t1, ssum, ssq = _t1(xw, agg, gWn[i], gb[i])
        h2, heat = _atten(gsum, x, adj, aWq[i], aWk[i], abq[i], abk[i],
                          sWs[i], sWn[i], sb[i], bn2_g[i], bn2_b[i])
        x, hsub = _combine(t1, ssum, ssq, bn_g[i], bn_b[i], h2)
    out = _readout(hsub, W1, b1, W2, b2, nt)
    return (out, heat)


# speedup vs baseline: 10.5178x; 1.2740x over previous
import functools
import math

import jax
import jax.numpy as jnp
from jax import lax
from jax.experimental import pallas as pl
from jax.experimental.pallas import tpu as pltpu
from jax.experimental.pallas import tpu_sc as plsc

G = 16
S = 40
NPS = 40
N = G * S * NPS
D = 128
DH = D // 2
E = 409600
ON = G * NPS
EO = 10240
GS = G * S

NTILES = 16
EPT = E // NTILES
CH = 128
NCH = EPT // CH
ZROWS = N // NTILES


DQ = D // 4


def _sc_segsum(x4, src_r, dst_r, zrows):
    mesh = plsc.VectorSubcoreMesh(core_axis_name="c", subcore_axis_name="s")

    @functools.partial(
        pl.kernel,
        out_type=jax.ShapeDtypeStruct((4, N, DQ), jnp.float32),
        mesh=mesh,
        scratch_types=[
            pltpu.VMEM((NCH, CH), jnp.int32),
            pltpu.VMEM((NCH, CH), jnp.int32),
            pltpu.VMEM((CH, DQ), jnp.float32),
            pltpu.VMEM((CH, DQ), jnp.float32),
            pltpu.VMEM((CH, DQ), jnp.float32),
            pltpu.VMEM((CH, DQ), jnp.float32),
            pltpu.VMEM_SHARED((N, DQ), jnp.float32),
        ] + [pltpu.SemaphoreType.DMA] * 8,
        compiler_params=pltpu.CompilerParams(use_tc_tiling_on_sc=False),
    )
    def segsum(x4_hbm, src_hbm, dst_hbm, z_hbm, out_hbm,
               srcv, dstv, r0, r1, r2, r3,
               acc, g0, g1, g2, g3, s0, s1, s2, s3):
        bufs = (r0, r1, r2, r3)
        gsems = (g0, g1, g2, g3)
        ssems = (s0, s1, s2, s3)
        c = lax.axis_index("c")
        s = lax.axis_index("s")

        pltpu.sync_copy(src_hbm.at[s], srcv)
        pltpu.sync_copy(dst_hbm.at[s], dstv)

        @pl.loop(0, NCH)
        def _(j):
            @pl.loop(0, CH, step=16)
            def _(k):
                v = srcv[j, pl.ds(k, 16)]
                srcv[j, pl.ds(k, 16)] = v * 4 + 2 * c

        for p in range(2):
            pltpu.sync_copy(z_hbm, acc.at[pl.ds(s * ZROWS, ZROWS)])
            plsc.subcore_barrier()

            for b in range(4):
                pltpu.async_copy(x4_hbm.at[srcv.at[b]], bufs[b], gsems[b])

            @pl.loop(0, NCH, step=4)
            def _(j):
                cps = []
                for b in range(4):
                    pltpu.make_async_copy(
                        x4_hbm.at[srcv.at[j + b]], bufs[b], gsems[b]).wait()
                    cps.append(pltpu.async_copy(
                        bufs[b], acc.at[dstv.at[j + b]], ssems[b], add=True))
                for b in range(4):
                    cps[b].wait()

                    @pl.when(j + b + 4 < NCH)
                    def _(b=b):
                        pltpu.async_copy(x4_hbm.at[srcv.at[j + b + 4]],
                                         bufs[b], gsems[b])

            plsc.subcore_barrier()
            pltpu.sync_copy(acc.at[pl.ds(s * ZROWS, ZROWS)],
                            out_hbm.at[2 * c + p, pl.ds(s * ZROWS, ZROWS)])

            if p == 0:
                plsc.subcore_barrier()

                @pl.loop(0, NCH)
                def _(j):
                    @pl.loop(0, CH, step=16)
                    def _(k):
                        srcv[j, pl.ds(k, 16)] = srcv[j, pl.ds(k, 16)] + 1

    return segsum(x4, src_r, dst_r, zrows)


BR = N // G


def _adj_kernel(s_ref, d_ref, adj_ref):
    i = pl.program_id(0)

    @pl.when(i == 0)
    def _():
        adj_ref[...] = jnp.zeros_like(adj_ref)

    cols = lax.broadcasted_iota(jnp.int32, (EO // 8, ON), 1)
    oh_s = (s_ref[0, 0, :][:, None] == cols).astype(jnp.bfloat16)
    oh_d = (d_ref[0, 0, :][:, None] == cols).astype(jnp.bfloat16)
    adj_ref[...] += lax.dot_general(
        oh_d, oh_s, (((0,), (0,)), ((), ())),
        preferred_element_type=jnp.float32)


def _build_adj(oe):
    src = oe[0].reshape(8, 1, EO // 8)
    dst = oe[1].reshape(8, 1, EO // 8)
    return pl.pallas_call(
        _adj_kernel,
        grid=(8,),
        in_specs=[
            pl.BlockSpec((1, 1, EO // 8), lambda i: (i, 0, 0)),
            pl.BlockSpec((1, 1, EO // 8), lambda i: (i, 0, 0)),
        ],
        out_specs=pl.BlockSpec((ON, ON), lambda i: (0, 0)),
        out_shape=jax.ShapeDtypeStruct((ON, ON), jnp.float32),
    )(src, dst)


def _xws_kernel(x_ref, w_ref, xw_ref, gsum_ref):
    xb = x_ref[...]
    xw_ref[...] = jnp.dot(xb, w_ref[...], preferred_element_type=jnp.float32)
    gsum_ref[...] = xb.reshape(S, NPS, D).sum(axis=1)


def _xws(x, w):
    return pl.pallas_call(
        _xws_kernel,
        grid=(G,),
        in_specs=[
            pl.BlockSpec((BR, D), lambda g: (g, 0)),
            pl.BlockSpec((D, D), lambda g: (0, 0)),
        ],
        out_specs=[
            pl.BlockSpec((BR, D), lambda g: (g, 0)),
            pl.BlockSpec((S, D), lambda g: (g, 0)),
        ],
        out_shape=[
            jax.ShapeDtypeStruct((N, D), jnp.float32),
            jax.ShapeDtypeStruct((GS, D), jnp.float32),
        ],
    )(x, w)


def _t1_kernel(xw_ref, a0_ref, a1_ref, a2_ref, a3_ref,
               w0_ref, w1_ref, w2_ref, w3_ref, b_ref,
               t1_ref, ssum_ref, ssq_ref):
    g = pl.program_id(0)
    t1 = xw_ref[...] + b_ref[...]
    for a_ref, w_ref in ((a0_ref, w0_ref), (a1_ref, w1_ref),
                         (a2_ref, w2_ref), (a3_ref, w3_ref)):
        t1 += jnp.dot(a_ref[0], w_ref[...], preferred_element_type=jnp.float32)
    t1_ref[...] = t1

    @pl.when(g == 0)
    def _():
        ssum_ref[...] = jnp.zeros_like(ssum_ref)
        ssq_ref[...] = jnp.zeros_like(ssq_ref)

    ssum_ref[...] += jnp.sum(t1, axis=0, keepdims=True)
    ssq_ref[...] += jnp.sum(t1 * t1, axis=0, keepdims=True)


def _t1(xw, agg, wn, b):
    return pl.pallas_call(
        _t1_kernel,
        grid=(G,),
        in_specs=(
            [pl.BlockSpec((BR, D), lambda g: (g, 0))]
            + [pl.BlockSpec((1, BR, DQ),
                            functools.partial(lambda q, g: (q, g, 0), q))
               for q in range(4)]
            + [pl.BlockSpec((DQ, D), lambda g: (0, 0)) for _ in range(4)]
            + [pl.BlockSpec((1, D), lambda g: (0, 0))]
        ),
        out_specs=[
            pl.BlockSpec((BR, D), lambda g: (g, 0)),
            pl.BlockSpec((1, D), lambda g: (0, 0)),
            pl.BlockSpec((1, D), lambda g: (0, 0)),
        ],
        out_shape=[
            jax.ShapeDtypeStruct((N, D), jnp.float32),
            jax.ShapeDtypeStruct((1, D), jnp.float32),
            jax.ShapeDtypeStruct((1, D), jnp.float32),
        ],
    )(xw, agg, agg, agg, agg,
      wn[0:DQ], wn[DQ:2 * DQ], wn[2 * DQ:3 * DQ], wn[3 * DQ:4 * DQ],
      b.reshape(1, D))


def _atten_kernel(gsum_ref, x_ref, adj_ref, wq_ref, wk_ref, bq_ref, bk_ref,
                  sws_ref, swn_ref, sb_ref, g2_ref, b2_ref,
                  h2_ref, heat_ref):
    gsf = gsum_ref[...] * (1.0 / NPS)
    q = jnp.dot(gsf, wq_ref[...], preferred_element_type=jnp.float32) + bq_ref[...]
    k = jnp.dot(gsf, wk_ref[...], preferred_element_type=jnp.float32) + bk_ref[...]
    scale = 1.0 / math.sqrt(float(D))
    xa_parts = []
    for g in range(G):
        qg = lax.slice(q, (g * S, 0), ((g + 1) * S, D))
        kg = lax.slice(k, (g * S, 0), ((g + 1) * S, D))
        sc = lax.dot_general(qg, kg, (((1,), (1,)), ((), ())),
                             preferred_element_type=jnp.float32) * scale
        m = jnp.max(sc, axis=-1, keepdims=True)
        ex = jnp.exp(sc - m)
        a = ex / jnp.sum(ex, axis=-1, keepdims=True)
        if g == G - 1:
            heat_ref[...] = a
        xg = x_ref[pl.ds(g * BR, BR), :].reshape(S, NPS, D)
        xa_parts.append(jnp.sum(a[:, :, None] * xg, axis=0))
    xa = jnp.concatenate(xa_parts, axis=0)
    aggo = jnp.dot(adj_ref[...].astype(jnp.float32), xa,
                   preferred_element_type=jnp.float32)
    pre = (jnp.dot(xa, sws_ref[...], preferred_element_type=jnp.float32)
           + jnp.dot(aggo, swn_ref[...], preferred_element_type=jnp.float32)
           + sb_ref[...])
    mu = jnp.mean(pre, axis=0, keepdims=True)
    var = jnp.mean((pre - mu) * (pre - mu), axis=0, keepdims=True)
    h2_ref[...] = ((pre - mu) * lax.rsqrt(var + 1e-5) * g2_ref[...]
                   + b2_ref[...])


def _atten(gsum, x, adj, wq, wk, bq, bk, sws, swn, sb, g2, b2):
    full = lambda shape: pl.BlockSpec(shape, lambda: tuple(0 for _ in shape))
    return pl.pallas_call(
        _atten_kernel,
        in_specs=[
            full((GS, D)), full((N, D)), full((ON, ON)),
            full((D, D)), full((D, D)), full((1, D)), full((1, D)),
            full((D, D)), full((D, D)), full((1, D)),
            full((1, D)), full((1, D)),
        ],
        out_specs=[full((ON, D)), full((S, S))],
        out_shape=[
            jax.ShapeDtypeStruct((ON, D), jnp.float32),
            jax.ShapeDtypeStruct((S, S), jnp.float32),
        ],
    )(gsum, x, adj, wq, wk, bq.reshape(1, D), bk.reshape(1, D),
      sws, swn, sb.reshape(1, D), g2.reshape(1, D), b2.reshape(1, D))


def _combine_kernel(t1_ref, ssum_ref, ssq_ref, g_ref, b_ref, h2_ref,
                    x_ref, hsub_ref):
    mu = ssum_ref[...] * (1.0 / N)
    var = ssq_ref[...] * (1.0 / N) - mu * mu
    sc = lax.rsqrt(var + 1e-5) * g_ref[...]
    sh = b_ref[...] - mu * sc
    t1 = t1_ref[...].reshape(S, NPS, D)
    xn = jnp.maximum(t1 * sc[0][None, None, :] + sh[0][None, None, :]
                     + h2_ref[...][None, :, :], 0.0)
    x_ref[...] = xn.reshape(BR, D)
    hsub_ref[...] = xn.sum(axis=1) * (1.0 / NPS)


def _combine(t1, ssum, ssq, g, b, h2):
    return pl.pallas_call(
        _combine_kernel,
        grid=(G,),
        in_specs=[
            pl.BlockSpec((BR, D), lambda g: (g, 0)),
            pl.BlockSpec((1, D), lambda g: (0, 0)),
            pl.BlockSpec((1, D), lambda g: (0, 0)),
            pl.BlockSpec((1, D), lambda g: (0, 0)),
            pl.BlockSpec((1, D), lambda g: (0, 0)),
            pl.BlockSpec((NPS, D), lambda g: (g, 0)),
        ],
        out_specs=[
            pl.BlockSpec((BR, D), lambda g: (g, 0)),
            pl.BlockSpec((S, D), lambda g: (g, 0)),
        ],
        out_shape=[
            jax.ShapeDtypeStruct((N, D), jnp.float32),
            jax.ShapeDtypeStruct((GS, D), jnp.float32),
        ],
    )(t1, ssum, ssq, g.reshape(1, D), b.reshape(1, D), h2)


def _readout_kernel(hsub_ref, w1_ref, b1_ref, w2_ref, b2_ref, out_ref):
    hg = hsub_ref[...].reshape(G, S, D).mean(axis=1)
    h = jnp.maximum(
        jnp.dot(hg, w1_ref[...], preferred_element_type=jnp.float32)
        + b1_ref[...], 0.0)
    out_ref[...] = (jnp.dot(h, w2_ref[...], preferred_element_type=jnp.float32)
                    + b2_ref[...])


def _readout(hsub, w1, b1, w2, b2, nt):
    full = lambda shape: pl.BlockSpec(shape, lambda: tuple(0 for _ in shape))
    return pl.pallas_call(
        _readout_kernel,
        in_specs=[full((GS, D)), full((D, 2 * D)), full((1, 2 * D)),
                  full((2 * D, nt)), full((1, nt))],
        out_specs=full((G, nt)),
        out_shape=jax.ShapeDtypeStruct((G, nt), jnp.float32),
    )(hsub, w1, b1.reshape(1, 2 * D), w2, b2.reshape(1, nt))


def kernel(x, edge_index, original_edge_index, batch, num_subgraphs,
           num_nodes_per_subgraph, subgraph_batch, subgraph_node_idx,
           subgraph_idx_batch, gWs, gWn, gb, bn_g, bn_b, sWs, sWn, sb,
           bn2_g, bn2_b, aWq, aWk, abq, abk, W1, b1, W2, b2):
    L = gWs.shape[0]
    nt = W2.shape[1]
    src_r = edge_index[0].astype(jnp.int32).reshape(NTILES, NCH, CH)
    dst_r = edge_index[1].astype(jnp.int32).reshape(NTILES, NCH, CH)
    zrows = jnp.zeros((ZROWS, DQ), jnp.float32)
    adj = _build_adj(original_edge_index.astype(jnp.int32))

    heat = None
    hsub = None
    for i in range(L):
        agg = _sc_segsum(x.reshape(4 * N, DQ), src_r, dst_r, zrows)
        xw, gsum = _xws(x, gWs[i])
        t1, ssum, ssq = _t1(xw, agg, gWn[i], gb[i])
        h2, heat = _atten(gsum, x, adj, aWq[i], aWk[i], abq[i], abk[i],
                          sWs[i], sWn[i], sb[i], bn2_g[i], bn2_b[i])
        x, hsub = _combine(t1, ssum, ssq, bn_g[i], bn_b[i], h2)
    out = _readout(hsub, W1, b1, W2, b2, nt)
    return (out, heat)


# speedup vs baseline: 12.2154x; 1.1614x over previous
import functools
import math

import jax
import jax.numpy as jnp
from jax import lax
from jax.experimental import pallas as pl
from jax.experimental.pallas import tpu as pltpu
from jax.experimental.pallas import tpu_sc as plsc

G = 16
S = 40
NPS = 40
N = G * S * NPS
D = 128
DH = D // 2
E = 409600
ON = G * NPS
EO = 10240
GS = G * S

NTILES = 16
EPT = E // NTILES
CH = 128
NCH = EPT // CH
ZROWS = N // NTILES


DQ = D // 4


def _sc_segsum(x4, src_r, dst_r, zrows):
    mesh = plsc.VectorSubcoreMesh(core_axis_name="c", subcore_axis_name="s")

    @functools.partial(
        pl.kernel,
        out_type=jax.ShapeDtypeStruct((N, D), jnp.float32),
        mesh=mesh,
        scratch_types=[
            pltpu.VMEM((NCH, CH), jnp.int32),
            pltpu.VMEM((NCH, CH), jnp.int32),
            pltpu.VMEM((CH, DQ), jnp.float32),
            pltpu.VMEM((CH, DQ), jnp.float32),
            pltpu.VMEM((CH, DQ), jnp.float32),
            pltpu.VMEM((CH, DQ), jnp.float32),
            pltpu.VMEM_SHARED((N, DQ), jnp.float32),
        ] + [pltpu.SemaphoreType.DMA] * 8,
        compiler_params=pltpu.CompilerParams(use_tc_tiling_on_sc=False),
    )
    def segsum(x4_hbm, src_hbm, dst_hbm, z_hbm, out_hbm,
               srcv, dstv, r0, r1, r2, r3,
               acc, g0, g1, g2, g3, s0, s1, s2, s3):
        bufs = (r0, r1, r2, r3)
        gsems = (g0, g1, g2, g3)
        ssems = (s0, s1, s2, s3)
        c = lax.axis_index("c")
        s = lax.axis_index("s")

        pltpu.sync_copy(src_hbm.at[s], srcv)
        pltpu.sync_copy(dst_hbm.at[s], dstv)

        @pl.loop(0, NCH)
        def _(j):
            @pl.loop(0, CH, step=16)
            def _(k):
                v = srcv[j, pl.ds(k, 16)]
                srcv[j, pl.ds(k, 16)] = v * 4 + 2 * c

        for p in range(2):
            pltpu.sync_copy(z_hbm, acc.at[pl.ds(s * ZROWS, ZROWS)])
            plsc.subcore_barrier()

            for b in range(4):
                pltpu.async_copy(x4_hbm.at[srcv.at[b]], bufs[b], gsems[b])

            @pl.loop(0, NCH, step=4)
            def _(j):
                cps = []
                for b in range(4):
                    pltpu.make_async_copy(
                        x4_hbm.at[srcv.at[j + b]], bufs[b], gsems[b]).wait()
                    cps.append(pltpu.async_copy(
                        bufs[b], acc.at[dstv.at[j + b]], ssems[b], add=True))
                for b in range(4):
                    cps[b].wait()

                    @pl.when(j + b + 4 < NCH)
                    def _(b=b):
                        pltpu.async_copy(x4_hbm.at[srcv.at[j + b + 4]],
                                         bufs[b], gsems[b])

            plsc.subcore_barrier()
            pltpu.sync_copy(
                acc.at[pl.ds(s * ZROWS, ZROWS)],
                out_hbm.at[pl.ds(s * ZROWS, ZROWS),
                           pl.ds((2 * c + p) * DQ, DQ)])

            if p == 0:
                plsc.subcore_barrier()

                @pl.loop(0, NCH)
                def _(j):
                    @pl.loop(0, CH, step=16)
                    def _(k):
                        srcv[j, pl.ds(k, 16)] = srcv[j, pl.ds(k, 16)] + 1

    return segsum(x4, src_r, dst_r, zrows)


BR = N // G


def _adj_kernel(s_ref, d_ref, adj_ref):
    i = pl.program_id(0)

    @pl.when(i == 0)
    def _():
        adj_ref[...] = jnp.zeros_like(adj_ref)

    cols = lax.broadcasted_iota(jnp.int32, (EO // 8, ON), 1)
    oh_s = (s_ref[0, 0, :][:, None] == cols).astype(jnp.bfloat16)
    oh_d = (d_ref[0, 0, :][:, None] == cols).astype(jnp.bfloat16)
    adj_ref[...] += lax.dot_general(
        oh_d, oh_s, (((0,), (0,)), ((), ())),
        preferred_element_type=jnp.float32)


def _build_adj(oe):
    src = oe[0].reshape(8, 1, EO // 8)
    dst = oe[1].reshape(8, 1, EO // 8)
    return pl.pallas_call(
        _adj_kernel,
        grid=(8,),
        in_specs=[
            pl.BlockSpec((1, 1, EO // 8), lambda i: (i, 0, 0)),
            pl.BlockSpec((1, 1, EO // 8), lambda i: (i, 0, 0)),
        ],
        out_specs=pl.BlockSpec((ON, ON), lambda i: (0, 0)),
        out_shape=jax.ShapeDtypeStruct((ON, ON), jnp.float32),
    )(src, dst)


def _xws_kernel(x_ref, w_ref, xw_ref, gsum_ref):
    xb = x_ref[...]
    xw_ref[...] = jnp.dot(xb, w_ref[...], preferred_element_type=jnp.float32)
    gsum_ref[...] = xb.reshape(S, NPS, D).sum(axis=1)


def _xws(x, w):
    return pl.pallas_call(
        _xws_kernel,
        grid=(G,),
        in_specs=[
            pl.BlockSpec((BR, D), lambda g: (g, 0)),
            pl.BlockSpec((D, D), lambda g: (0, 0)),
        ],
        out_specs=[
            pl.BlockSpec((BR, D), lambda g: (g, 0)),
            pl.BlockSpec((S, D), lambda g: (g, 0)),
        ],
        out_shape=[
            jax.ShapeDtypeStruct((N, D), jnp.float32),
            jax.ShapeDtypeStruct((GS, D), jnp.float32),
        ],
    )(x, w)


def _t1_kernel(xw_ref, agg_ref, wn_ref, b_ref, t1_ref, ssum_ref, ssq_ref):
    g = pl.program_id(0)
    t1 = (xw_ref[...] + b_ref[...]
          + jnp.dot(agg_ref[...], wn_ref[...],
                    preferred_element_type=jnp.float32))
    t1_ref[...] = t1

    @pl.when(g == 0)
    def _():
        ssum_ref[...] = jnp.zeros_like(ssum_ref)
        ssq_ref[...] = jnp.zeros_like(ssq_ref)

    ssum_ref[...] += jnp.sum(t1, axis=0, keepdims=True)
    ssq_ref[...] += jnp.sum(t1 * t1, axis=0, keepdims=True)


def _t1(xw, agg, wn, b):
    return pl.pallas_call(
        _t1_kernel,
        grid=(G,),
        in_specs=[
            pl.BlockSpec((BR, D), lambda g: (g, 0)),
            pl.BlockSpec((BR, D), lambda g: (g, 0)),
            pl.BlockSpec((D, D), lambda g: (0, 0)),
            pl.BlockSpec((1, D), lambda g: (0, 0)),
        ],
        out_specs=[
            pl.BlockSpec((BR, D), lambda g: (g, 0)),
            pl.BlockSpec((1, D), lambda g: (0, 0)),
            pl.BlockSpec((1, D), lambda g: (0, 0)),
        ],
        out_shape=[
            jax.ShapeDtypeStruct((N, D), jnp.float32),
            jax.ShapeDtypeStruct((1, D), jnp.float32),
            jax.ShapeDtypeStruct((1, D), jnp.float32),
        ],
    )(xw, agg, wn, b.reshape(1, D))


def _atten_kernel(gsum_ref, x_ref, adj_ref, wq_ref, wk_ref, bq_ref, bk_ref,
                  sws_ref, swn_ref, sb_ref, g2_ref, b2_ref,
                  h2_ref, heat_ref):
    gsf = gsum_ref[...] * (1.0 / NPS)
    q = jnp.dot(gsf, wq_ref[...], preferred_element_type=jnp.float32) + bq_ref[...]
    k = jnp.dot(gsf, wk_ref[...], preferred_element_type=jnp.float32) + bk_ref[...]
    scale = 1.0 / math.sqrt(float(D))
    qk = lax.dot_general(q, k, (((1,), (1,)), ((), ())),
                         preferred_element_type=jnp.float32)
    xa_parts = []
    for g in range(G):
        sc = lax.slice(qk, (g * S, g * S), ((g + 1) * S, (g + 1) * S)) * scale
        m = jnp.max(sc, axis=-1, keepdims=True)
        ex = jnp.exp(sc - m)
        a = ex / jnp.sum(ex, axis=-1, keepdims=True)
        if g == G - 1:
            heat_ref[...] = a
        xg = x_ref[pl.ds(g * BR, BR), :].reshape(S, NPS, D)
        xa_parts.append(jnp.sum(a[:, :, None] * xg, axis=0))
    xa = jnp.concatenate(xa_parts, axis=0)
    aggo = jnp.dot(adj_ref[...].astype(jnp.float32), xa,
                   preferred_element_type=jnp.float32)
    pre = (jnp.dot(xa, sws_ref[...], preferred_element_type=jnp.float32)
           + jnp.dot(aggo, swn_ref[...], preferred_element_type=jnp.float32)
           + sb_ref[...])
    mu = jnp.mean(pre, axis=0, keepdims=True)
    var = jnp.mean((pre - mu) * (pre - mu), axis=0, keepdims=True)
    h2_ref[...] = ((pre - mu) * lax.rsqrt(var + 1e-5) * g2_ref[...]
                   + b2_ref[...])


def _atten(gsum, x, adj, wq, wk, bq, bk, sws, swn, sb, g2, b2):
    full = lambda shape: pl.BlockSpec(shape, lambda: tuple(0 for _ in shape))
    return pl.pallas_call(
        _atten_kernel,
        in_specs=[
            full((GS, D)), full((N, D)), full((ON, ON)),
            full((D, D)), full((D, D)), full((1, D)), full((1, D)),
            full((D, D)), full((D, D)), full((1, D)),
            full((1, D)), full((1, D)),
        ],
        out_specs=[full((ON, D)), full((S, S))],
        out_shape=[
            jax.ShapeDtypeStruct((ON, D), jnp.float32),
            jax.ShapeDtypeStruct((S, S), jnp.float32),
        ],
    )(gsum, x, adj, wq, wk, bq.reshape(1, D), bk.reshape(1, D),
      sws, swn, sb.reshape(1, D), g2.reshape(1, D), b2.reshape(1, D))


def _combine_kernel(t1_ref, ssum_ref, ssq_ref, g_ref, b_ref, h2_ref,
                    x_ref, hsub_ref):
    mu = ssum_ref[...] * (1.0 / N)
    var = ssq_ref[...] * (1.0 / N) - mu * mu
    sc = lax.rsqrt(var + 1e-5) * g_ref[...]
    sh = b_ref[...] - mu * sc
    t1 = t1_ref[...].reshape(S, NPS, D)
    xn = jnp.maximum(t1 * sc[0][None, None, :] + sh[0][None, None, :]
                     + h2_ref[...][None, :, :], 0.0)
    x_ref[...] = xn.reshape(BR, D)
    hsub_ref[...] = xn.sum(axis=1) * (1.0 / NPS)


def _combine(t1, ssum, ssq, g, b, h2):
    return pl.pallas_call(
        _combine_kernel,
        grid=(G,),
        in_specs=[
            pl.BlockSpec((BR, D), lambda g: (g, 0)),
            pl.BlockSpec((1, D), lambda g: (0, 0)),
            pl.BlockSpec((1, D), lambda g: (0, 0)),
            pl.BlockSpec((1, D), lambda g: (0, 0)),
            pl.BlockSpec((1, D), lambda g: (0, 0)),
            pl.BlockSpec((NPS, D), lambda g: (g, 0)),
        ],
        out_specs=[
            pl.BlockSpec((BR, D), lambda g: (g, 0)),
            pl.BlockSpec((S, D), lambda g: (g, 0)),
        ],
        out_shape=[
            jax.ShapeDtypeStruct((N, D), jnp.float32),
            jax.ShapeDtypeStruct((GS, D), jnp.float32),
        ],
    )(t1, ssum, ssq, g.reshape(1, D), b.reshape(1, D), h2)


def _readout_kernel(hsub_ref, w1_ref, b1_ref, w2_ref, b2_ref, out_ref):
    hg = hsub_ref[...].reshape(G, S, D).mean(axis=1)
    h = jnp.maximum(
        jnp.dot(hg, w1_ref[...], preferred_element_type=jnp.float32)
        + b1_ref[...], 0.0)
    out_ref[...] = (jnp.dot(h, w2_ref[...], preferred_element_type=jnp.float32)
                    + b2_ref[...])


def _readout(hsub, w1, b1, w2, b2, nt):
    full = lambda shape: pl.BlockSpec(shape, lambda: tuple(0 for _ in shape))
    return pl.pallas_call(
        _readout_kernel,
        in_specs=[full((GS, D)), full((D, 2 * D)), full((1, 2 * D)),
                  full((2 * D, nt)), full((1, nt))],
        out_specs=full((G, nt)),
        out_shape=jax.ShapeDtypeStruct((G, nt), jnp.float32),
    )(hsub, w1, b1.reshape(1, 2 * D), w2, b2.reshape(1, nt))


def kernel(x, edge_index, original_edge_index, batch, num_subgraphs,
           num_nodes_per_subgraph, subgraph_batch, subgraph_node_idx,
           subgraph_idx_batch, gWs, gWn, gb, bn_g, bn_b, sWs, sWn, sb,
           bn2_g, bn2_b, aWq, aWk, abq, abk, W1, b1, W2, b2):
    L = gWs.shape[0]
    nt = W2.shape[1]
    src_r = edge_index[0].astype(jnp.int32).reshape(NTILES, NCH, CH)
    dst_r = edge_index[1].astype(jnp.int32).reshape(NTILES, NCH, CH)
    zrows = jnp.zeros((ZROWS, DQ), jnp.float32)
    adj = _build_adj(original_edge_index.astype(jnp.int32))

    heat = None
    hsub = None
    for i in range(L):
        agg = _sc_segsum(x.reshape(4 * N, DQ), src_r, dst_r, zrows)
        xw, gsum = _xws(x, gWs[i])
        t1, ssum, ssq = _t1(xw, agg, gWn[i], gb[i])
        h2, heat = _atten(gsum, x, adj, aWq[i], aWk[i], abq[i], abk[i],
                          sWs[i], sWn[i], sb[i], bn2_g[i], bn2_b[i])
        x, hsub = _combine(t1, ssum, ssq, bn_g[i], bn_b[i], h2)
    out = _readout(hsub, W1, b1, W2, b2, nt)
    return (out, heat)


# speedup vs baseline: 12.7367x; 1.0427x over previous
import functools
import math

import jax
import jax.numpy as jnp
from jax import lax
from jax.experimental import pallas as pl
from jax.experimental.pallas import tpu as pltpu
from jax.experimental.pallas import tpu_sc as plsc

G = 16
S = 40
NPS = 40
N = G * S * NPS
D = 128
DH = D // 2
E = 409600
ON = G * NPS
EO = 10240
GS = G * S

NTILES = 16
EPT = E // NTILES
CH = 128
NCH = EPT // CH
ZROWS = N // NTILES


DQ = D // 4


def _sc_segsum(x4, src_r, dst_r, zrows):
    mesh = plsc.VectorSubcoreMesh(core_axis_name="c", subcore_axis_name="s")

    @functools.partial(
        pl.kernel,
        out_type=jax.ShapeDtypeStruct((N, D), jnp.float32),
        mesh=mesh,
        scratch_types=[
            pltpu.VMEM((NCH, CH), jnp.int32),
            pltpu.VMEM((NCH, CH), jnp.int32),
            pltpu.VMEM((CH, DQ), jnp.float32),
            pltpu.VMEM((CH, DQ), jnp.float32),
            pltpu.VMEM((CH, DQ), jnp.float32),
            pltpu.VMEM((CH, DQ), jnp.float32),
            pltpu.VMEM_SHARED((N, DQ), jnp.float32),
        ] + [pltpu.SemaphoreType.DMA] * 8,
        compiler_params=pltpu.CompilerParams(use_tc_tiling_on_sc=False),
    )
    def segsum(x4_hbm, src_hbm, dst_hbm, z_hbm, out_hbm,
               srcv, dstv, r0, r1, r2, r3,
               acc, g0, g1, g2, g3, s0, s1, s2, s3):
        bufs = (r0, r1, r2, r3)
        gsems = (g0, g1, g2, g3)
        ssems = (s0, s1, s2, s3)
        c = lax.axis_index("c")
        s = lax.axis_index("s")

        pltpu.sync_copy(src_hbm.at[s], srcv)
        pltpu.sync_copy(dst_hbm.at[s], dstv)

        @pl.loop(0, NCH)
        def _(j):
            @pl.loop(0, CH, step=16)
            def _(k):
                v = srcv[j, pl.ds(k, 16)]
                srcv[j, pl.ds(k, 16)] = v * 4 + 2 * c

        for p in range(2):
            pltpu.sync_copy(z_hbm, acc.at[pl.ds(s * ZROWS, ZROWS)])
            plsc.subcore_barrier()

            for b in range(4):
                pltpu.async_copy(x4_hbm.at[srcv.at[b]], bufs[b], gsems[b])

            @pl.loop(0, NCH, step=4)
            def _(j):
                cps = []
                for b in range(4):
                    pltpu.make_async_copy(
                        x4_hbm.at[srcv.at[j + b]], bufs[b], gsems[b]).wait()
                    cps.append(pltpu.async_copy(
                        bufs[b], acc.at[dstv.at[j + b]], ssems[b], add=True))
                for b in range(4):
                    cps[b].wait()

                    @pl.when(j + b + 4 < NCH)
                    def _(b=b):
                        pltpu.async_copy(x4_hbm.at[srcv.at[j + b + 4]],
                                         bufs[b], gsems[b])

            plsc.subcore_barrier()
            pltpu.sync_copy(
                acc.at[pl.ds(s * ZROWS, ZROWS)],
                out_hbm.at[pl.ds(s * ZROWS, ZROWS),
                           pl.ds((2 * c + p) * DQ, DQ)])

            if p == 0:
                plsc.subcore_barrier()

                @pl.loop(0, NCH)
                def _(j):
                    @pl.loop(0, CH, step=16)
                    def _(k):
                        srcv[j, pl.ds(k, 16)] = srcv[j, pl.ds(k, 16)] + 1

    return segsum(x4, src_r, dst_r, zrows)


GPB = 2
BR = GPB * S * NPS
NB = G // GPB


def _adj_kernel(s_ref, d_ref, adj_ref):
    i = pl.program_id(0)

    @pl.when(i == 0)
    def _():
        adj_ref[...] = jnp.zeros_like(adj_ref)

    cols = lax.broadcasted_iota(jnp.int32, (EO // 8, ON), 1)
    oh_s = (s_ref[0, 0, :][:, None] == cols).astype(jnp.bfloat16)
    oh_d = (d_ref[0, 0, :][:, None] == cols).astype(jnp.bfloat16)
    adj_ref[...] += lax.dot_general(
        oh_d, oh_s, (((0,), (0,)), ((), ())),
        preferred_element_type=jnp.float32)


def _build_adj(oe):
    src = oe[0].reshape(8, 1, EO // 8)
    dst = oe[1].reshape(8, 1, EO // 8)
    return pl.pallas_call(
        _adj_kernel,
        grid=(8,),
        in_specs=[
            pl.BlockSpec((1, 1, EO // 8), lambda i: (i, 0, 0)),
            pl.BlockSpec((1, 1, EO // 8), lambda i: (i, 0, 0)),
        ],
        out_specs=pl.BlockSpec((ON, ON), lambda i: (0, 0)),
        out_shape=jax.ShapeDtypeStruct((ON, ON), jnp.float32),
    )(src, dst)


def _xws_kernel(x_ref, w_ref, xw_ref, gsum_ref):
    xb = x_ref[...]
    xw_ref[...] = jnp.dot(xb, w_ref[...], preferred_element_type=jnp.float32)
    gsum_ref[...] = xb.reshape(GPB * S, NPS, D).sum(axis=1)


def _xws(x, w):
    return pl.pallas_call(
        _xws_kernel,
        grid=(NB,),
        in_specs=[
            pl.BlockSpec((BR, D), lambda g: (g, 0)),
            pl.BlockSpec((D, D), lambda g: (0, 0)),
        ],
        out_specs=[
            pl.BlockSpec((BR, D), lambda g: (g, 0)),
            pl.BlockSpec((GPB * S, D), lambda g: (g, 0)),
        ],
        out_shape=[
            jax.ShapeDtypeStruct((N, D), jnp.float32),
            jax.ShapeDtypeStruct((GS, D), jnp.float32),
        ],
    )(x, w)


def _t1_kernel(xw_ref, agg_ref, wn_ref, b_ref, t1_ref, ssum_ref, ssq_ref):
    g = pl.program_id(0)
    t1 = (xw_ref[...] + b_ref[...]
          + jnp.dot(agg_ref[...], wn_ref[...],
                    preferred_element_type=jnp.float32))
    t1_ref[...] = t1

    @pl.when(g == 0)
    def _():
        ssum_ref[...] = jnp.zeros_like(ssum_ref)
        ssq_ref[...] = jnp.zeros_like(ssq_ref)

    ssum_ref[...] += jnp.sum(t1, axis=0, keepdims=True)
    ssq_ref[...] += jnp.sum(t1 * t1, axis=0, keepdims=True)


def _t1(xw, agg, wn, b):
    return pl.pallas_call(
        _t1_kernel,
        grid=(NB,),
        in_specs=[
            pl.BlockSpec((BR, D), lambda g: (g, 0)),
            pl.BlockSpec((BR, D), lambda g: (g, 0)),
            pl.BlockSpec((D, D), lambda g: (0, 0)),
            pl.BlockSpec((1, D), lambda g: (0, 0)),
        ],
        out_specs=[
            pl.BlockSpec((BR, D), lambda g: (g, 0)),
            pl.BlockSpec((1, D), lambda g: (0, 0)),
            pl.BlockSpec((1, D), lambda g: (0, 0)),
        ],
        out_shape=[
            jax.ShapeDtypeStruct((N, D), jnp.float32),
            jax.ShapeDtypeStruct((1, D), jnp.float32),
            jax.ShapeDtypeStruct((1, D), jnp.float32),
        ],
    )(xw, agg, wn, b.reshape(1, D))


def _atten_kernel(gsum_ref, x_ref, adj_ref, wq_ref, wk_ref, bq_ref, bk_ref,
                  sws_ref, swn_ref, sb_ref, g2_ref, b2_ref,
                  h2_ref, heat_ref):
    gsf = gsum_ref[...] * (1.0 / NPS)
    q = jnp.dot(gsf, wq_ref[...], preferred_element_type=jnp.float32) + bq_ref[...]
    k = jnp.dot(gsf, wk_ref[...], preferred_element_type=jnp.float32) + bk_ref[...]
    scale = 1.0 / math.sqrt(float(D))
    qk = lax.dot_general(q, k, (((1,), (1,)), ((), ())),
                         preferred_element_type=jnp.float32)
    xa_parts = []
    for g in range(G):
        sc = lax.slice(qk, (g * S, g * S), ((g + 1) * S, (g + 1) * S)) * scale
        m = jnp.max(sc, axis=-1, keepdims=True)
        ex = jnp.exp(sc - m)
        a = ex / jnp.sum(ex, axis=-1, keepdims=True)
        if g == G - 1:
            heat_ref[...] = a
        xg = x_ref[pl.ds(g * S * NPS, S * NPS), :].reshape(S, NPS, D)
        xa_parts.append(jnp.sum(a[:, :, None] * xg, axis=0))
    xa = jnp.concatenate(xa_parts, axis=0)
    aggo = jnp.dot(adj_ref[...].astype(jnp.float32), xa,
                   preferred_element_type=jnp.float32)
    pre = (jnp.dot(xa, sws_ref[...], preferred_element_type=jnp.float32)
           + jnp.dot(aggo, swn_ref[...], preferred_element_type=jnp.float32)
           + sb_ref[...])
    mu = jnp.mean(pre, axis=0, keepdims=True)
    var = jnp.mean((pre - mu) * (pre - mu), axis=0, keepdims=True)
    h2_ref[...] = ((pre - mu) * lax.rsqrt(var + 1e-5) * g2_ref[...]
                   + b2_ref[...])


def _atten(gsum, x, adj, wq, wk, bq, bk, sws, swn, sb, g2, b2):
    full = lambda shape: pl.BlockSpec(shape, lambda: tuple(0 for _ in shape))
    return pl.pallas_call(
        _atten_kernel,
        in_specs=[
            full((GS, D)), full((N, D)), full((ON, ON)),
            full((D, D)), full((D, D)), full((1, D)), full((1, D)),
            full((D, D)), full((D, D)), full((1, D)),
            full((1, D)), full((1, D)),
        ],
        out_specs=[full((ON, D)), full((S, S))],
        out_shape=[
            jax.ShapeDtypeStruct((ON, D), jnp.float32),
            jax.ShapeDtypeStruct((S, S), jnp.float32),
        ],
    )(gsum, x, adj, wq, wk, bq.reshape(1, D), bk.reshape(1, D),
      sws, swn, sb.reshape(1, D), g2.reshape(1, D), b2.reshape(1, D))


def _combine_kernel(t1_ref, ssum_ref, ssq_ref, g_ref, b_ref, h2_ref,
                    x_ref, hsub_ref):
    mu = ssum_ref[...] * (1.0 / N)
    var = ssq_ref[...] * (1.0 / N) - mu * mu
    sc = lax.rsqrt(var + 1e-5) * g_ref[...]
    sh = b_ref[...] - mu * sc
    t1 = t1_ref[...].reshape(GPB, S, NPS, D)
    h2b = h2_ref[...].reshape(GPB, 1, NPS, D)
    xn = jnp.maximum(t1 * sc[0][None, None, None, :]
                     + sh[0][None, None, None, :] + h2b, 0.0)
    x_ref[...] = xn.reshape(BR, D)
    hsub_ref[...] = (xn.sum(axis=2) * (1.0 / NPS)).reshape(GPB * S, D)


def _combine(t1, ssum, ssq, g, b, h2):
    return pl.pallas_call(
        _combine_kernel,
        grid=(NB,),
        in_specs=[
            pl.BlockSpec((BR, D), lambda g: (g, 0)),
            pl.BlockSpec((1, D), lambda g: (0, 0)),
            pl.BlockSpec((1, D), lambda g: (0, 0)),
            pl.BlockSpec((1, D), lambda g: (0, 0)),
            pl.BlockSpec((1, D), lambda g: (0, 0)),
            pl.BlockSpec((GPB * NPS, D), lambda g: (g, 0)),
        ],
        out_specs=[
            pl.BlockSpec((BR, D), lambda g: (g, 0)),
            pl.BlockSpec((GPB * S, D), lambda g: (g, 0)),
        ],
        out_shape=[
            jax.ShapeDtypeStruct((N, D), jnp.float32),
            jax.ShapeDtypeStruct((GS, D), jnp.float32),
        ],
    )(t1, ssum, ssq, g.reshape(1, D), b.reshape(1, D), h2)


def _readout_kernel(hsub_ref, w1_ref, b1_ref, w2_ref, b2_ref, out_ref):
    hg = hsub_ref[...].reshape(G, S, D).mean(axis=1)
    h = jnp.maximum(
        jnp.dot(hg, w1_ref[...], preferred_element_type=jnp.float32)
        + b1_ref[...], 0.0)
    out_ref[...] = (jnp.dot(h, w2_ref[...], preferred_element_type=jnp.float32)
                    + b2_ref[...])


def _readout(hsub, w1, b1, w2, b2, nt):
    full = lambda shape: pl.BlockSpec(shape, lambda: tuple(0 for _ in shape))
    return pl.pallas_call(
        _readout_kernel,
        in_specs=[full((GS, D)), full((D, 2 * D)), full((1, 2 * D)),
                  full((2 * D, nt)), full((1, nt))],
        out_specs=full((G, nt)),
        out_shape=jax.ShapeDtypeStruct((G, nt), jnp.float32),
    )(hsub, w1, b1.reshape(1, 2 * D), w2, b2.reshape(1, nt))


def kernel(x, edge_index, original_edge_index, batch, num_subgraphs,
           num_nodes_per_subgraph, subgraph_batch, subgraph_node_idx,
           subgraph_idx_batch, gWs, gWn, gb, bn_g, bn_b, sWs, sWn, sb,
           bn2_g, bn2_b, aWq, aWk, abq, abk, W1, b1, W2, b2):
    L = gWs.shape[0]
    nt = W2.shape[1]
    src_r = edge_index[0].astype(jnp.int32).reshape(NTILES, NCH, CH)
    dst_r = edge_index[1].astype(jnp.int32).reshape(NTILES, NCH, CH)
    zrows = jnp.zeros((ZROWS, DQ), jnp.float32)
    adj = _build_adj(original_edge_index.astype(jnp.int32))

    heat = None
    hsub = None
    for i in range(L):
        agg = _sc_segsum(x.reshape(4 * N, DQ), src_r, dst_r, zrows)
        xw, gsum = _xws(x, gWs[i])
        t1, ssum, ssq = _t1(xw, agg, gWn[i], gb[i])
        h2, heat = _atten(gsum, x, adj, aWq[i], aWk[i], abq[i], abk[i],
                          sWs[i], sWn[i], sb[i], bn2_g[i], bn2_b[i])
        x, hsub = _combine(t1, ssum, ssq, bn_g[i], bn_b[i], h2)
    out = _readout(hsub, W1, b1, W2, b2, nt)
    return (out, heat)


# speedup vs baseline: 12.8104x; 1.0058x over previous
import functools
import math

import jax
import jax.numpy as jnp
from jax import lax
from jax.experimental import pallas as pl
from jax.experimental.pallas import tpu as pltpu
from jax.experimental.pallas import tpu_sc as plsc

G = 16
S = 40
NPS = 40
N = G * S * NPS
D = 128
DH = D // 2
E = 409600
ON = G * NPS
EO = 10240
GS = G * S

NTILES = 16
EPT = E // NTILES
CH = 128
NCH = EPT // CH
ZROWS = N // NTILES


DQ = D // 4


def _sc_segsum(x4, src_r, dst_r, zrows):
    mesh = plsc.VectorSubcoreMesh(core_axis_name="c", subcore_axis_name="s")

    @functools.partial(
        pl.kernel,
        out_type=jax.ShapeDtypeStruct((N, D), jnp.float32),
        mesh=mesh,
        scratch_types=[
            pltpu.VMEM((NCH, CH), jnp.int32),
            pltpu.VMEM((NCH, CH), jnp.int32),
            pltpu.VMEM((CH, DQ), jnp.float32),
            pltpu.VMEM((CH, DQ), jnp.float32),
            pltpu.VMEM((CH, DQ), jnp.float32),
            pltpu.VMEM((CH, DQ), jnp.float32),
            pltpu.VMEM_SHARED((N, DQ), jnp.float32),
        ] + [pltpu.SemaphoreType.DMA] * 8,
        compiler_params=pltpu.CompilerParams(use_tc_tiling_on_sc=False),
    )
    def segsum(x4_hbm, src_hbm, dst_hbm, z_hbm, out_hbm,
               srcv, dstv, r0, r1, r2, r3,
               acc, g0, g1, g2, g3, s0, s1, s2, s3):
        bufs = (r0, r1, r2, r3)
        gsems = (g0, g1, g2, g3)
        ssems = (s0, s1, s2, s3)
        c = lax.axis_index("c")
        s = lax.axis_index("s")

        pltpu.sync_copy(src_hbm.at[s], srcv)
        pltpu.sync_copy(dst_hbm.at[s], dstv)

        @pl.loop(0, NCH)
        def _(j):
            @pl.loop(0, CH, step=16)
            def _(k):
                v = srcv[j, pl.ds(k, 16)]
                srcv[j, pl.ds(k, 16)] = v * 4 + 2 * c

        for p in range(2):
            pltpu.sync_copy(z_hbm, acc.at[pl.ds(s * ZROWS, ZROWS)])
            plsc.subcore_barrier()

            for b in range(4):
                pltpu.async_copy(x4_hbm.at[srcv.at[b]], bufs[b], gsems[b])

            @pl.loop(0, NCH, step=4)
            def _(j):
                cps = []
                for b in range(4):
                    pltpu.make_async_copy(
                        x4_hbm.at[srcv.at[j + b]], bufs[b], gsems[b]).wait()
                    cps.append(pltpu.async_copy(
                        bufs[b], acc.at[dstv.at[j + b]], ssems[b], add=True))
                for b in range(4):
                    cps[b].wait()

                    @pl.when(j + b + 4 < NCH)
                    def _(b=b):
                        pltpu.async_copy(x4_hbm.at[srcv.at[j + b + 4]],
                                         bufs[b], gsems[b])

            plsc.subcore_barrier()
            pltpu.sync_copy(
                acc.at[pl.ds(s * ZROWS, ZROWS)],
                out_hbm.at[pl.ds(s * ZROWS, ZROWS),
                           pl.ds((2 * c + p) * DQ, DQ)])

            if p == 0:
                plsc.subcore_barrier()

                @pl.loop(0, NCH)
                def _(j):
                    @pl.loop(0, CH, step=16)
                    def _(k):
                        srcv[j, pl.ds(k, 16)] = srcv[j, pl.ds(k, 16)] + 1

    return segsum(x4, src_r, dst_r, zrows)


GPB = 4
BR = GPB * S * NPS
NB = G // GPB


def _adj_kernel(s_ref, d_ref, adj_ref):
    i = pl.program_id(0)

    @pl.when(i == 0)
    def _():
        adj_ref[...] = jnp.zeros_like(adj_ref)

    cols = lax.broadcasted_iota(jnp.int32, (EO // 8, ON), 1)
    oh_s = (s_ref[0, 0, :][:, None] == cols).astype(jnp.bfloat16)
    oh_d = (d_ref[0, 0, :][:, None] == cols).astype(jnp.bfloat16)
    adj_ref[...] += lax.dot_general(
        oh_d, oh_s, (((0,), (0,)), ((), ())),
        preferred_element_type=jnp.float32)


def _build_adj(oe):
    src = oe[0].reshape(8, 1, EO // 8)
    dst = oe[1].reshape(8, 1, EO // 8)
    return pl.pallas_call(
        _adj_kernel,
        grid=(8,),
        in_specs=[
            pl.BlockSpec((1, 1, EO // 8), lambda i: (i, 0, 0)),
            pl.BlockSpec((1, 1, EO // 8), lambda i: (i, 0, 0)),
        ],
        out_specs=pl.BlockSpec((ON, ON), lambda i: (0, 0)),
        out_shape=jax.ShapeDtypeStruct((ON, ON), jnp.float32),
    )(src, dst)


def _xws_kernel(x_ref, w_ref, xw_ref, gsum_ref):
    xb = x_ref[...]
    xw_ref[...] = jnp.dot(xb, w_ref[...], preferred_element_type=jnp.float32)
    gsum_ref[...] = xb.reshape(GPB * S, NPS, D).sum(axis=1)


def _xws(x, w):
    return pl.pallas_call(
        _xws_kernel,
        grid=(NB,),
        in_specs=[
            pl.BlockSpec((BR, D), lambda g: (g, 0)),
            pl.BlockSpec((D, D), lambda g: (0, 0)),
        ],
        out_specs=[
            pl.BlockSpec((BR, D), lambda g: (g, 0)),
            pl.BlockSpec((GPB * S, D), lambda g: (g, 0)),
        ],
        out_shape=[
            jax.ShapeDtypeStruct((N, D), jnp.float32),
            jax.ShapeDtypeStruct((GS, D), jnp.float32),
        ],
    )(x, w)


def _t1_kernel(xw_ref, agg_ref, wn_ref, b_ref, t1_ref, ssum_ref, ssq_ref):
    g = pl.program_id(0)
    t1 = (xw_ref[...] + b_ref[...]
          + jnp.dot(agg_ref[...], wn_ref[...],
                    preferred_element_type=jnp.float32))
    t1_ref[...] = t1

    @pl.when(g == 0)
    def _():
        ssum_ref[...] = jnp.zeros_like(ssum_ref)
        ssq_ref[...] = jnp.zeros_like(ssq_ref)

    ssum_ref[...] += jnp.sum(t1, axis=0, keepdims=True)
    ssq_ref[...] += jnp.sum(t1 * t1, axis=0, keepdims=True)


def _t1(xw, agg, wn, b):
    return pl.pallas_call(
        _t1_kernel,
        grid=(NB,),
        in_specs=[
            pl.BlockSpec((BR, D), lambda g: (g, 0)),
            pl.BlockSpec((BR, D), lambda g: (g, 0)),
            pl.BlockSpec((D, D), lambda g: (0, 0)),
            pl.BlockSpec((1, D), lambda g: (0, 0)),
        ],
        out_specs=[
            pl.BlockSpec((BR, D), lambda g: (g, 0)),
            pl.BlockSpec((1, D), lambda g: (0, 0)),
            pl.BlockSpec((1, D), lambda g: (0, 0)),
        ],
        out_shape=[
            jax.ShapeDtypeStruct((N, D), jnp.float32),
            jax.ShapeDtypeStruct((1, D), jnp.float32),
            jax.ShapeDtypeStruct((1, D), jnp.float32),
        ],
    )(xw, agg, wn, b.reshape(1, D))


def _atten_kernel(gsum_ref, x_ref, adj_ref, wq_ref, wk_ref, bq_ref, bk_ref,
                  sws_ref, swn_ref, sb_ref, g2_ref, b2_ref,
                  h2_ref, heat_ref):
    gsf = gsum_ref[...] * (1.0 / NPS)
    q = jnp.dot(gsf, wq_ref[...], preferred_element_type=jnp.float32) + bq_ref[...]
    k = jnp.dot(gsf, wk_ref[...], preferred_element_type=jnp.float32) + bk_ref[...]
    scale = 1.0 / math.sqrt(float(D))
    qk = lax.dot_general(q, k, (((1,), (1,)), ((), ())),
                         preferred_element_type=jnp.float32)
    xa_parts = []
    for g in range(G):
        sc = lax.slice(qk, (g * S, g * S), ((g + 1) * S, (g + 1) * S)) * scale
        m = jnp.max(sc, axis=-1, keepdims=True)
        ex = jnp.exp(sc - m)
        a = ex / jnp.sum(ex, axis=-1, keepdims=True)
        if g == G - 1:
            heat_ref[...] = a
        xg = x_ref[pl.ds(g * S * NPS, S * NPS), :].reshape(S, NPS, D)
        xa_parts.append(jnp.sum(a[:, :, None] * xg, axis=0))
    xa = jnp.concatenate(xa_parts, axis=0)
    aggo = jnp.dot(adj_ref[...].astype(jnp.float32), xa,
                   preferred_element_type=jnp.float32)
    pre = (jnp.dot(xa, sws_ref[...], preferred_element_type=jnp.float32)
           + jnp.dot(aggo, swn_ref[...], preferred_element_type=jnp.float32)
           + sb_ref[...])
    mu = jnp.mean(pre, axis=0, keepdims=True)
    var = jnp.mean((pre - mu) * (pre - mu), axis=0, keepdims=True)
    h2_ref[...] = ((pre - mu) * lax.rsqrt(var + 1e-5) * g2_ref[...]
                   + b2_ref[...])


def _atten(gsum, x, adj, wq, wk, bq, bk, sws, swn, sb, g2, b2):
    full = lambda shape: pl.BlockSpec(shape, lambda: tuple(0 for _ in shape))
    return pl.pallas_call(
        _atten_kernel,
        in_specs=[
            full((GS, D)), full((N, D)), full((ON, ON)),
            full((D, D)), full((D, D)), full((1, D)), full((1, D)),
            full((D, D)), full((D, D)), full((1, D)),
            full((1, D)), full((1, D)),
        ],
        out_specs=[full((ON, D)), full((S, S))],
        out_shape=[
            jax.ShapeDtypeStruct((ON, D), jnp.float32),
            jax.ShapeDtypeStruct((S, S), jnp.float32),
        ],
    )(gsum, x, adj, wq, wk, bq.reshape(1, D), bk.reshape(1, D),
      sws, swn, sb.reshape(1, D), g2.reshape(1, D), b2.reshape(1, D))


def _combine_kernel(t1_ref, ssum_ref, ssq_ref, g_ref, b_ref, h2_ref,
                    x_ref, hsub_ref):
    mu = ssum_ref[...] * (1.0 / N)
    var = ssq_ref[...] * (1.0 / N) - mu * mu
    sc = lax.rsqrt(var + 1e-5) * g_ref[...]
    sh = b_ref[...] - mu * sc
    t1 = t1_ref[...].reshape(GPB, S, NPS, D)
    h2b = h2_ref[...].reshape(GPB, 1, NPS, D)
    xn = jnp.maximum(t1 * sc[0][None, None, None, :]
                     + sh[0][None, None, None, :] + h2b, 0.0)
    x_ref[...] = xn.reshape(BR, D)
    hsub_ref[...] = (xn.sum(axis=2) * (1.0 / NPS)).reshape(GPB * S, D)


def _combine(t1, ssum, ssq, g, b, h2):
    return pl.pallas_call(
        _combine_kernel,
        grid=(NB,),
        in_specs=[
            pl.BlockSpec((BR, D), lambda g: (g, 0)),
            pl.BlockSpec((1, D), lambda g: (0, 0)),
            pl.BlockSpec((1, D), lambda g: (0, 0)),
            pl.BlockSpec((1, D), lambda g: (0, 0)),
            pl.BlockSpec((1, D), lambda g: (0, 0)),
            pl.BlockSpec((GPB * NPS, D), lambda g: (g, 0)),
        ],
        out_specs=[
            pl.BlockSpec((BR, D), lambda g: (g, 0)),
            pl.BlockSpec((GPB * S, D), lambda g: (g, 0)),
        ],
        out_shape=[
            jax.ShapeDtypeStruct((N, D), jnp.float32),
            jax.ShapeDtypeStruct((GS, D), jnp.float32),
        ],
    )(t1, ssum, ssq, g.reshape(1, D), b.reshape(1, D), h2)


def _readout_kernel(hsub_ref, w1_ref, b1_ref, w2_ref, b2_ref, out_ref):
    hg = hsub_ref[...].reshape(G, S, D).mean(axis=1)
    h = jnp.maximum(
        jnp.dot(hg, w1_ref[...], preferred_element_type=jnp.float32)
        + b1_ref[...], 0.0)
    out_ref[...] = (jnp.dot(h, w2_ref[...], preferred_element_type=jnp.float32)
                    + b2_ref[...])


def _readout(hsub, w1, b1, w2, b2, nt):
    full = lambda shape: pl.BlockSpec(shape, lambda: tuple(0 for _ in shape))
    return pl.pallas_call(
        _readout_kernel,
        in_specs=[full((GS, D)), full((D, 2 * D)), full((1, 2 * D)),
                  full((2 * D, nt)), full((1, nt))],
        out_specs=full((G, nt)),
        out_shape=jax.ShapeDtypeStruct((G, nt), jnp.float32),
    )(hsub, w1, b1.reshape(1, 2 * D), w2, b2.reshape(1, nt))


def kernel(x, edge_index, original_edge_index, batch, num_subgraphs,
           num_nodes_per_subgraph, subgraph_batch, subgraph_node_idx,
           subgraph_idx_batch, gWs, gWn, gb, bn_g, bn_b, sWs, sWn, sb,
           bn2_g, bn2_b, aWq, aWk, abq, abk, W1, b1, W2, b2):
    L = gWs.shape[0]
    nt = W2.shape[1]
    src_r = edge_index[0].astype(jnp.int32).reshape(NTILES, NCH, CH)
    dst_r = edge_index[1].astype(jnp.int32).reshape(NTILES, NCH, CH)
    zrows = jnp.zeros((ZROWS, DQ), jnp.float32)
    adj = _build_adj(original_edge_index.astype(jnp.int32))

    heat = None
    hsub = None
    for i in range(L):
        agg = _sc_segsum(x.reshape(4 * N, DQ), src_r, dst_r, zrows)
        xw, gsum = _xws(x, gWs[i])
        t1, ssum, ssq = _t1(xw, agg, gWn[i], gb[i])
        h2, heat = _atten(gsum, x, adj, aWq[i], aWk[i], abq[i], abk[i],
                          sWs[i], sWn[i], sb[i], bn2_g[i], bn2_b[i])
        x, hsub = _combine(t1, ssum, ssq, bn_g[i], bn_b[i], h2)
    out = _readout(hsub, W1, b1, W2, b2, nt)
    return (out, heat)
